# Initial kernel scaffold; baseline (speedup 1.0000x reference)
import functools

import jax
import jax.numpy as jnp
from jax.experimental import pallas as pl

B, K, L = 1, 512, 256
VOCAB, C_M, C_Z = 5, 256, 128
D_EMB = C_Z // 2
MAXREL = 32
VPAD = 8
RELPAD = 72
TOK_BLK = 4096
L_BLK = 16


def _msa_body(tok_ref, tab_ref, out_ref):
    t = jnp.broadcast_to(tok_ref[...], (TOK_BLK, VPAD))
    lane = jax.lax.broadcasted_iota(jnp.int32, (TOK_BLK, VPAD), 1)
    onehot = (t == lane).astype(jnp.float32)
    out_ref[...] = jnp.dot(onehot, tab_ref[...],
                           preferred_element_type=jnp.float32)


def _pair_body(ptok_ref, ptab_ref, wl_ref, wr_ref, b_ref, rel_ref, out_ref):
    i = pl.program_id(0)
    t = jnp.broadcast_to(ptok_ref[...], (L, VPAD))
    lane = jax.lax.broadcasted_iota(jnp.int32, (L, VPAD), 1)
    onehot = (t == lane).astype(jnp.float32)
    seq_emb = jnp.dot(onehot, ptab_ref[...], preferred_element_type=jnp.float32)
    rpb = jnp.dot(seq_emb, wr_ref[...], preferred_element_type=jnp.float32)
    rpb = rpb + b_ref[...]
    se_blk = jax.lax.dynamic_slice_in_dim(seq_emb, i * L_BLK, L_BLK, 0)
    lp = jnp.dot(se_blk, wl_ref[...], preferred_element_type=jnp.float32)
    r = jax.lax.broadcasted_iota(jnp.int32, (L_BLK * L, RELPAD), 0)
    lane2 = jax.lax.broadcasted_iota(jnp.int32, (L_BLK * L, RELPAD), 1)
    m = r & (L - 1)
    lrow = (r >> 8) + i * L_BLK
    s = jnp.clip(m - lrow, -MAXREL, MAXREL) + MAXREL
    ohr = (s == lane2).astype(jnp.float32)
    rp = jnp.dot(ohr, rel_ref[...], preferred_element_type=jnp.float32)
    out_ref[...] = (rp.reshape(L_BLK, L, C_Z)
                    + lp[:, None, :] + rpb[None, :, :])


@jax.jit
def kernel(tokens, pair_tokens, msa_table, pair_table, proj_W, proj_b,
           relpos_table):
    tok_flat = tokens.reshape(K * L, 1).astype(jnp.int32)
    msa_pad = jnp.zeros((VPAD, C_M), jnp.float32).at[:VOCAB].set(msa_table)
    ptok = pair_tokens.reshape(L, 1).astype(jnp.int32)
    ptab = jnp.zeros((VPAD, D_EMB), jnp.float32).at[:VOCAB].set(pair_table)
    wl_t = proj_W[:, :D_EMB].T
    wr_t = proj_W[:, D_EMB:].T
    b2 = proj_b.reshape(1, C_Z)
    rel_pad = jnp.zeros((RELPAD, C_Z), jnp.float32).at[:2 * MAXREL + 1].set(
        relpos_table)

    msa_flat = pl.pallas_call(
        _msa_body,
        grid=(K * L // TOK_BLK,),
        in_specs=[
            pl.BlockSpec((TOK_BLK, 1), lambda i: (i, 0)),
            pl.BlockSpec((VPAD, C_M), lambda i: (0, 0)),
        ],
        out_specs=pl.BlockSpec((TOK_BLK, C_M), lambda i: (i, 0)),
        out_shape=jax.ShapeDtypeStruct((K * L, C_M), jnp.float32),
    )(tok_flat, msa_pad)

    pair = pl.pallas_call(
        _pair_body,
        grid=(L // L_BLK,),
        in_specs=[
            pl.BlockSpec((L, 1), lambda i: (0, 0)),
            pl.BlockSpec((VPAD, D_EMB), lambda i: (0, 0)),
            pl.BlockSpec((D_EMB, C_Z), lambda i: (0, 0)),
            pl.BlockSpec((D_EMB, C_Z), lambda i: (0, 0)),
            pl.BlockSpec((1, C_Z), lambda i: (0, 0)),
            pl.BlockSpec((RELPAD, C_Z), lambda i: (0, 0)),
        ],
        out_specs=pl.BlockSpec((L_BLK, L, C_Z), lambda i: (i, 0, 0)),
        out_shape=jax.ShapeDtypeStruct((L, L, C_Z), jnp.float32),
    )(ptok, ptab, wl_t, wr_t, b2, rel_pad)

    return (msa_flat.reshape(B, K, L, C_M), pair.reshape(B, L, L, C_Z))


# speedup vs baseline: 6.5899x; 6.5899x over previous
import functools

import jax
import jax.numpy as jnp
from jax.experimental import pallas as pl

B, K, L = 1, 512, 256
VOCAB, C_M, C_Z = 5, 256, 128
D_EMB = C_Z // 2
MAXREL = 32
VPAD = 8
RELPAD = 72
TOK_BLK = 4096
L_BLK = 16


def _msa_body(tok_ref, tab_ref, out_ref):
    t = jnp.broadcast_to(tok_ref[...], (TOK_BLK, VPAD))
    lane = jax.lax.broadcasted_iota(jnp.int32, (TOK_BLK, VPAD), 1)
    onehot = (t == lane).astype(jnp.float32)
    out_ref[...] = jnp.dot(onehot, tab_ref[...],
                           preferred_element_type=jnp.float32)


def _pair_body(ptok_ref, ptok_blk_ref, ptab_ref, wl_ref, wr_ref, b_ref,
               rel_ref, out_ref):
    i = pl.program_id(0)
    t = jnp.broadcast_to(ptok_ref[...], (L, VPAD))
    lane = jax.lax.broadcasted_iota(jnp.int32, (L, VPAD), 1)
    onehot = (t == lane).astype(jnp.float32)
    seq_emb = jnp.dot(onehot, ptab_ref[...], preferred_element_type=jnp.float32)
    rpb = jnp.dot(seq_emb, wr_ref[...], preferred_element_type=jnp.float32)
    rpb = rpb + b_ref[...]
    tb = jnp.broadcast_to(ptok_blk_ref[...], (L_BLK, VPAD))
    lane_b = jax.lax.broadcasted_iota(jnp.int32, (L_BLK, VPAD), 1)
    se_blk = jnp.dot((tb == lane_b).astype(jnp.float32), ptab_ref[...],
                     preferred_element_type=jnp.float32)
    lp = jnp.dot(se_blk, wl_ref[...], preferred_element_type=jnp.float32)
    r = jax.lax.broadcasted_iota(jnp.int32, (L_BLK * L, RELPAD), 0)
    lane2 = jax.lax.broadcasted_iota(jnp.int32, (L_BLK * L, RELPAD), 1)
    m = r & (L - 1)
    lrow = (r >> 8) + i * L_BLK
    s = jnp.clip(m - lrow, -MAXREL, MAXREL) + MAXREL
    ohr = (s == lane2).astype(jnp.float32)
    rp = jnp.dot(ohr, rel_ref[...], preferred_element_type=jnp.float32)
    out_ref[...] = (rp.reshape(L_BLK, L, C_Z)
                    + lp[:, None, :] + rpb[None, :, :])


@jax.jit
def kernel(tokens, pair_tokens, msa_table, pair_table, proj_W, proj_b,
           relpos_table):
    tok_flat = tokens.reshape(K * L, 1).astype(jnp.int32)
    msa_pad = jnp.zeros((VPAD, C_M), jnp.float32).at[:VOCAB].set(msa_table)
    ptok = pair_tokens.reshape(L, 1).astype(jnp.int32)
    ptab = jnp.zeros((VPAD, D_EMB), jnp.float32).at[:VOCAB].set(pair_table)
    wl_t = proj_W[:, :D_EMB].T
    wr_t = proj_W[:, D_EMB:].T
    b2 = proj_b.reshape(1, C_Z)
    rel_pad = jnp.zeros((RELPAD, C_Z), jnp.float32).at[:2 * MAXREL + 1].set(
        relpos_table)

    msa_flat = pl.pallas_call(
        _msa_body,
        grid=(K * L // TOK_BLK,),
        in_specs=[
            pl.BlockSpec((TOK_BLK, 1), lambda i: (i, 0)),
            pl.BlockSpec((VPAD, C_M), lambda i: (0, 0)),
        ],
        out_specs=pl.BlockSpec((TOK_BLK, C_M), lambda i: (i, 0)),
        out_shape=jax.ShapeDtypeStruct((K * L, C_M), jnp.float32),
    )(tok_flat, msa_pad)

    pair = pl.pallas_call(
        _pair_body,
        grid=(L // L_BLK,),
        in_specs=[
            pl.BlockSpec((L, 1), lambda i: (0, 0)),
            pl.BlockSpec((L_BLK, 1), lambda i: (i, 0)),
            pl.BlockSpec((VPAD, D_EMB), lambda i: (0, 0)),
            pl.BlockSpec((D_EMB, C_Z), lambda i: (0, 0)),
            pl.BlockSpec((D_EMB, C_Z), lambda i: (0, 0)),
            pl.BlockSpec((1, C_Z), lambda i: (0, 0)),
            pl.BlockSpec((RELPAD, C_Z), lambda i: (0, 0)),
        ],
        out_specs=pl.BlockSpec((L_BLK, L, C_Z), lambda i: (i, 0, 0)),
        out_shape=jax.ShapeDtypeStruct((L, L, C_Z), jnp.float32),
    )(ptok, ptok, ptab, wl_t, wr_t, b2, rel_pad)

    return (msa_flat.reshape(B, K, L, C_M), pair.reshape(B, L, L, C_Z))


# speedup vs baseline: 11.0880x; 1.6826x over previous
import functools

import jax
import jax.numpy as jnp
from jax.experimental import pallas as pl

B, K, L = 1, 512, 256
VOCAB, C_M, C_Z = 5, 256, 128
D_EMB = C_Z // 2
MAXREL = 32
VPAD = 8
RELPAD = 72
TOK_BLK = 4096
L_BLK = 16


def _msa_body(tok_ref, tab_ref, out_ref):
    t = jnp.broadcast_to(tok_ref[...].reshape(1, TOK_BLK), (VPAD, TOK_BLK))
    vrow = jax.lax.broadcasted_iota(jnp.int32, (VPAD, TOK_BLK), 0)
    onehot_t = (t == vrow).astype(jnp.float32)
    out_ref[...] = jax.lax.dot_general(
        onehot_t, tab_ref[...], (((0,), (0,)), ((), ())),
        preferred_element_type=jnp.float32)


def _pair_body(ptok_ref, ptok_blk_ref, ptab_ref, wl_ref, wr_ref, b_ref,
               rel_ref, out_ref):
    i = pl.program_id(0)
    t = jnp.broadcast_to(ptok_ref[...].reshape(1, L), (VPAD, L))
    vrow = jax.lax.broadcasted_iota(jnp.int32, (VPAD, L), 0)
    onehot_t = (t == vrow).astype(jnp.float32)
    seq_emb = jax.lax.dot_general(
        onehot_t, ptab_ref[...], (((0,), (0,)), ((), ())),
        preferred_element_type=jnp.float32)
    rpb = jnp.dot(seq_emb, wr_ref[...], preferred_element_type=jnp.float32)
    rpb = rpb + b_ref[...]
    tb = jnp.broadcast_to(ptok_blk_ref[...], (L_BLK, VPAD))
    lane_b = jax.lax.broadcasted_iota(jnp.int32, (L_BLK, VPAD), 1)
    se_blk = jnp.dot((tb == lane_b).astype(jnp.float32), ptab_ref[...],
                     preferred_element_type=jnp.float32)
    lp = jnp.dot(se_blk, wl_ref[...], preferred_element_type=jnp.float32)
    r = jax.lax.broadcasted_iota(jnp.int32, (L_BLK * L, RELPAD), 0)
    lane2 = jax.lax.broadcasted_iota(jnp.int32, (L_BLK * L, RELPAD), 1)
    m = r & (L - 1)
    lrow = (r >> 8) + i * L_BLK
    s = jnp.clip(m - lrow, -MAXREL, MAXREL) + MAXREL
    ohr = (s == lane2).astype(jnp.float32)
    rp = jnp.dot(ohr, rel_ref[...], preferred_element_type=jnp.float32)
    out_ref[...] = (rp.reshape(L_BLK, L, C_Z)
                    + lp[:, None, :] + rpb[None, :, :])


@jax.jit
def kernel(tokens, pair_tokens, msa_table, pair_table, proj_W, proj_b,
           relpos_table):
    tok_flat = tokens.reshape(K * L // TOK_BLK, 1, TOK_BLK).astype(jnp.int32)
    msa_pad = jnp.zeros((VPAD, C_M), jnp.float32).at[:VOCAB].set(msa_table)
    ptok_w = pair_tokens.reshape(1, 1, L).astype(jnp.int32)
    ptok_blk = pair_tokens.reshape(L, 1).astype(jnp.int32)
    ptab = jnp.zeros((VPAD, D_EMB), jnp.float32).at[:VOCAB].set(pair_table)
    wl_t = proj_W[:, :D_EMB].T
    wr_t = proj_W[:, D_EMB:].T
    b2 = proj_b.reshape(1, C_Z)
    rel_pad = jnp.zeros((RELPAD, C_Z), jnp.float32).at[:2 * MAXREL + 1].set(
        relpos_table)

    msa_flat = pl.pallas_call(
        _msa_body,
        grid=(K * L // TOK_BLK,),
        in_specs=[
            pl.BlockSpec((1, 1, TOK_BLK), lambda i: (i, 0, 0)),
            pl.BlockSpec((VPAD, C_M), lambda i: (0, 0)),
        ],
        out_specs=pl.BlockSpec((TOK_BLK, C_M), lambda i: (i, 0)),
        out_shape=jax.ShapeDtypeStruct((K * L, C_M), jnp.float32),
    )(tok_flat, msa_pad)

    pair = pl.pallas_call(
        _pair_body,
        grid=(L // L_BLK,),
        in_specs=[
            pl.BlockSpec((1, 1, L), lambda i: (0, 0, 0)),
            pl.BlockSpec((L_BLK, 1), lambda i: (i, 0)),
            pl.BlockSpec((VPAD, D_EMB), lambda i: (0, 0)),
            pl.BlockSpec((D_EMB, C_Z), lambda i: (0, 0)),
            pl.BlockSpec((D_EMB, C_Z), lambda i: (0, 0)),
            pl.BlockSpec((1, C_Z), lambda i: (0, 0)),
            pl.BlockSpec((RELPAD, C_Z), lambda i: (0, 0)),
        ],
        out_specs=pl.BlockSpec((L_BLK, L, C_Z), lambda i: (i, 0, 0)),
        out_shape=jax.ShapeDtypeStruct((L, L, C_Z), jnp.float32),
    )(ptok_w, ptok_blk, ptab, wl_t, wr_t, b2, rel_pad)

    return (msa_flat.reshape(B, K, L, C_M), pair.reshape(B, L, L, C_Z))
